# Initial kernel scaffold; baseline (speedup 1.0000x reference)
import jax
import jax.numpy as jnp
from jax.experimental import pallas as pl


def kernel(x0, x1, x2, W_agg0, b_agg0, W_self0, W_agg1, b_agg1, W_self1):
    raise NotImplementedError("write your pallas kernel here")


# speedup vs baseline: 4.8600x; 4.8600x over previous
import functools

import jax
import jax.numpy as jnp
from jax.experimental import pallas as pl
from jax.experimental.pallas import tpu as pltpu

B = 5000
FANOUT = 10
D_IN = 128
D_H = 256
S = 40


def _gcn_body(x0_ref, x1_ref, x2_ref, wa0_ref, ba0_ref, ws0_ref,
              wa1_ref, ba1_ref, ws1_ref, out_ref):
    wa0 = wa0_ref[...]
    ws0 = ws0_ref[...]
    ba0 = ba0_ref[...]

    x2 = x2_ref[...]
    m2 = x2.reshape(S * FANOUT, FANOUT, D_IN).sum(axis=1) * (1.0 / FANOUT)
    x1 = x1_ref[...]
    h1 = (jnp.dot(x1, ws0, preferred_element_type=jnp.float32)
          + jnp.dot(m2, wa0, preferred_element_type=jnp.float32) + ba0)
    h1 = jnp.where(h1 >= 0, h1, 0.01 * h1)

    mh1 = h1.reshape(S, FANOUT, D_H).sum(axis=1) * (1.0 / FANOUT)
    m1 = x1.reshape(S, FANOUT, D_IN).sum(axis=1) * (1.0 / FANOUT)
    x0 = x0_ref[...]
    h0 = (jnp.dot(x0, ws0, preferred_element_type=jnp.float32)
          + jnp.dot(m1, wa0, preferred_element_type=jnp.float32) + ba0)
    h0 = jnp.where(h0 >= 0, h0, 0.01 * h0)

    out_ref[...] = (jnp.dot(h0, ws1_ref[...], preferred_element_type=jnp.float32)
                    + jnp.dot(mh1, wa1_ref[...], preferred_element_type=jnp.float32)
                    + ba1_ref[...])


@jax.jit
def kernel(x0, x1, x2, W_agg0, b_agg0, W_self0, W_agg1, b_agg1, W_self1):
    grid = (B // S,)
    full = lambda shape: pl.BlockSpec(shape, lambda i: (0,) * len(shape))
    return pl.pallas_call(
        _gcn_body,
        grid=grid,
        in_specs=[
            pl.BlockSpec((S, D_IN), lambda i: (i, 0)),
            pl.BlockSpec((S * FANOUT, D_IN), lambda i: (i, 0)),
            pl.BlockSpec((S * FANOUT * FANOUT, D_IN), lambda i: (i, 0)),
            full((D_IN, D_H)),
            full((1, D_H)),
            full((D_IN, D_H)),
            full((D_H, D_H)),
            full((1, D_H)),
            full((D_H, D_H)),
        ],
        out_specs=pl.BlockSpec((S, D_H), lambda i: (i, 0)),
        out_shape=jax.ShapeDtypeStruct((B, D_H), jnp.float32),
        compiler_params=pltpu.CompilerParams(
            dimension_semantics=("arbitrary",),
        ),
    )(x0, x1, x2, W_agg0, b_agg0.reshape(1, D_H), W_self0,
      W_agg1, b_agg1.reshape(1, D_H), W_self1)
